# baseline (speedup 1.0000x reference)
import functools

import jax
import jax.numpy as jnp
from jax import lax
from jax.experimental import pallas as pl
from jax.experimental.pallas import tpu as pltpu
from jax.experimental.pallas import tpu_sc as plsc

B = 16384
D = 32
H1 = 64
H2 = 32
NU = 100000
NS = 1000000
PACK = 4
B4 = B // PACK

_info = plsc.get_sparse_core_info()
_NC, _NSUB = _info.num_cores, _info.num_subcores
_NW = _NC * _NSUB
_BPW = B // _NW
_OPW = _BPW // PACK

_sc_mesh = plsc.VectorSubcoreMesh(core_axis_name="c", subcore_axis_name="s")


def _extract_rows(idx_v, pad_v, out_v):
    lane = lax.iota(jnp.int32, 16)

    def body(g, _):
        rows16 = g * 16 + lane
        off16 = plsc.load_gather(idx_v, [rows16])
        off16 = lax.shift_left(
            lax.bitwise_and(off16, jnp.full((16,), 3, jnp.int32)),
            jnp.full((16,), 5, jnp.int32),
        )
        dbase = rows16 * D
        for j in range(D):
            v = plsc.load_gather(pad_v, [rows16, off16 + j])
            dflat = dbase + j
            plsc.store_scatter(
                out_v,
                [lax.shift_right_logical(dflat, jnp.full((16,), 7, jnp.int32)),
                 lax.bitwise_and(dflat, jnp.full((16,), 127, jnp.int32))],
                v,
            )
        return 0

    lax.fori_loop(0, _BPW // 16, body, 0)


@functools.partial(
    pl.kernel,
    out_type=[
        jax.ShapeDtypeStruct((B4, 128), jnp.float32),
        jax.ShapeDtypeStruct((B4, 128), jnp.float32),
    ],
    mesh=_sc_mesh,
    compiler_params=pltpu.CompilerParams(needs_layout_passes=False),
    scratch_types=[
        pltpu.VMEM((_BPW,), jnp.int32),
        pltpu.VMEM((_BPW,), jnp.int32),
        pltpu.VMEM((_BPW,), jnp.int32),
        pltpu.VMEM((_BPW, 128), jnp.float32),
        pltpu.VMEM((_OPW, 128), jnp.float32),
        pltpu.VMEM((_OPW, 128), jnp.float32),
        pltpu.SemaphoreType.DMA,
    ],
)
def _sc_gather(u4_hbm, s4_hbm, uidx_hbm, sidx_hbm, uout_hbm, sout_hbm,
               uidx_v, sidx_v, p_v, pad_v, uout_v, sout_v, sem):
    wid = lax.axis_index("s") * _NC + lax.axis_index("c")
    base = wid * _BPW
    pltpu.sync_copy(uidx_hbm.at[pl.ds(base, _BPW)], uidx_v)
    pltpu.sync_copy(sidx_hbm.at[pl.ds(base, _BPW)], sidx_v)

    def padded_ids(idx_v):
        def pbody(t, _):
            sl = pl.ds(t * 16, 16)
            p_v[sl] = lax.shift_right_logical(
                idx_v[sl], jnp.full((16,), 2, jnp.int32))
            return 0

        lax.fori_loop(0, _BPW // 16, pbody, 0)

    padded_ids(uidx_v)
    pltpu.async_copy(u4_hbm.at[p_v], pad_v, sem).wait()
    _extract_rows(uidx_v, pad_v, uout_v)
    padded_ids(sidx_v)
    pltpu.async_copy(s4_hbm.at[p_v], pad_v, sem).wait()
    _extract_rows(sidx_v, pad_v, sout_v)

    obase = wid * _OPW
    pltpu.sync_copy(uout_v, uout_hbm.at[pl.ds(obase, _OPW)])
    pltpu.sync_copy(sout_v, sout_hbm.at[pl.ds(obase, _OPW)])


def _mlp_body(u_ref, s_ref, w_ref, t_ref, w1u_ref, w1s_ref, ww_ref, wt_ref,
              b1_ref, w2_ref, b2_ref, w3_ref, b3_ref, out_ref):
    x = (
        jnp.dot(u_ref[...], w1u_ref[...], preferred_element_type=jnp.float32)
        + jnp.dot(s_ref[...], w1s_ref[...], preferred_element_type=jnp.float32)
        + jnp.dot(w_ref[...], ww_ref[...], preferred_element_type=jnp.float32)
        + jnp.dot(t_ref[...], wt_ref[...], preferred_element_type=jnp.float32)
        + b1_ref[...]
    )
    h = jnp.maximum(x, 0.0)
    h = jnp.dot(h, w2_ref[...], preferred_element_type=jnp.float32) + b2_ref[...]
    h = jnp.maximum(h, 0.0)
    o = jnp.dot(h, w3_ref[...], preferred_element_type=jnp.float32) + b3_ref[...]
    out_ref[...] = jax.nn.sigmoid(o)


_BLK4 = 1024


def _mlp(u4, s4, w4, t4, w1u, w1s, ww, wt, b1, w2, b2, w3, b3):
    grid = (B4 // _BLK4,)
    return pl.pallas_call(
        _mlp_body,
        grid=grid,
        in_specs=[
            pl.BlockSpec((_BLK4, 128), lambda i: (i, 0)),
            pl.BlockSpec((_BLK4, 128), lambda i: (i, 0)),
            pl.BlockSpec((_BLK4, PACK), lambda i: (i, 0)),
            pl.BlockSpec((_BLK4, PACK), lambda i: (i, 0)),
            pl.BlockSpec((128, PACK * H1), lambda i: (0, 0)),
            pl.BlockSpec((128, PACK * H1), lambda i: (0, 0)),
            pl.BlockSpec((PACK, PACK * H1), lambda i: (0, 0)),
            pl.BlockSpec((PACK, PACK * H1), lambda i: (0, 0)),
            pl.BlockSpec((1, PACK * H1), lambda i: (0, 0)),
            pl.BlockSpec((PACK * H1, PACK * H2), lambda i: (0, 0)),
            pl.BlockSpec((1, PACK * H2), lambda i: (0, 0)),
            pl.BlockSpec((PACK * H2, PACK), lambda i: (0, 0)),
            pl.BlockSpec((1, PACK), lambda i: (0, 0)),
        ],
        out_specs=pl.BlockSpec((_BLK4, PACK), lambda i: (i, 0)),
        out_shape=jax.ShapeDtypeStruct((B4, PACK), jnp.float32),
    )(u4, s4, w4, t4, w1u, w1s, ww, wt, b1, w2, b2, w3, b3)


def _block_diag(m, k):
    a, b = m.shape
    out = jnp.zeros((k * a, k * b), m.dtype)
    for i in range(k):
        out = lax.dynamic_update_slice(out, m, (i * a, i * b))
    return out


@jax.jit
def kernel(user, song, weather, time, user_emb, song_emb, W1, b1, W2, b2, W3, b3):
    uidx = user.astype(jnp.int32)
    sidx = song.astype(jnp.int32)
    u4t = user_emb.reshape(NU * D // 128, 128)
    s4t = song_emb.reshape(NS * D // 128, 128)
    u4, s4 = _sc_gather(u4t, s4t, uidx, sidx)

    w4 = weather.reshape(B4, PACK)
    t4 = time.reshape(B4, PACK)

    w1u = _block_diag(W1[:D], PACK)
    w1s = _block_diag(W1[D:2 * D], PACK)
    ww = _block_diag(W1[2 * D][None, :], PACK)
    wt = _block_diag(W1[2 * D + 1][None, :], PACK)
    b1b = jnp.tile(b1, PACK)[None, :]
    w2b = _block_diag(W2, PACK)
    b2b = jnp.tile(b2, PACK)[None, :]
    w3b = _block_diag(W3, PACK)
    b3b = jnp.tile(b3, PACK)[None, :]

    out4 = _mlp(u4, s4, w4, t4, w1u, w1s, ww, wt, b1b, w2b, b2b, w3b, b3b)
    return out4.reshape(B)


# speedup vs baseline: 1.9901x; 1.9901x over previous
import functools

import jax
import jax.numpy as jnp
from jax import lax
from jax.experimental import pallas as pl
from jax.experimental.pallas import tpu as pltpu
from jax.experimental.pallas import tpu_sc as plsc

B = 16384
D = 32
H1 = 64
H2 = 32
NU = 100000
NS = 1000000

_info = plsc.get_sparse_core_info()
_NC, _NSUB = _info.num_cores, _info.num_subcores
_NW = _NC * _NSUB

_CHUNK = 2048
_RING = 4

_U_BLKS = NU // 128
_S_BLKS = NS // 128
_U_BOUND = _U_BLKS * 128
_S_BOUND = _S_BLKS * 128

_sc_mesh = plsc.VectorSubcoreMesh(core_axis_name="c", subcore_axis_name="s")


def _gather_table(tbl_hbm, out_hbm, idx_v, hcol_v, hpos_v, chunk_v, stage_v,
                  pos_v, sem, wid, nblks_total, nchunks):
    nbase = nblks_total // _NW
    nrem = nblks_total - nbase * _NW
    nblk = nbase + jnp.where(wid < nrem, 1, 0)
    blk0 = wid * nbase + jnp.minimum(wid, nrem)
    lo = blk0 * 128
    hi = lo + nblk * 128

    lane = lax.iota(jnp.int32, 16)
    lo_v = jnp.full((16,), 0, jnp.int32) + lo
    hi_v = jnp.full((16,), 0, jnp.int32) + hi

    def pf_body(g, hcnt):
        v16 = idx_v[pl.ds(g * 16, 16)]
        m = (v16 >= lo_v) & (v16 < hi_v)
        plsc.store_compressed(hcol_v.at[pl.ds(hcnt, 16)], v16, mask=m)
        plsc.store_compressed(
            hpos_v.at[pl.ds(hcnt, 16)], g * 16 + lane, mask=m)
        return hcnt + jnp.max(plsc.all_reduce_population_count(m))

    hcnt = lax.fori_loop(0, B // 16, pf_body, jnp.int32(0))
    hpos_v[pl.ds(hcnt, 16)] = jnp.full((16,), -1, jnp.int32)
    ngroups = (hcnt + 15) // 16

    def chunk_body(c, rc):
        start = jnp.minimum(lo + c * _CHUNK, hi - _CHUNK)
        pltpu.sync_copy(tbl_hbm.at[:, pl.ds(start, _CHUNK)], chunk_v)
        start_v = jnp.full((16,), 0, jnp.int32) + start

        def grp_body(g, rc):
            c16 = hcol_v[pl.ds(g * 16, 16)] - start_v
            p16 = hpos_v[pl.ds(g * 16, 16)]
            inm = (c16 >= 0) & (c16 < _CHUNK) & (p16 >= 0)
            nhit = jnp.max(plsc.all_reduce_population_count(inm))

            def do_extract(rc):
                slot = lax.bitwise_and(rc, jnp.int32(_RING - 1))

                @pl.when(rc >= _RING)
                def _():
                    pltpu.make_async_copy(
                        out_hbm.at[pl.ds(0, 16)],
                        stage_v.at[pl.ds(slot * 16, 16)], sem).wait()

                c16c = jnp.clip(c16, 0, _CHUNK - 1)
                rows16 = slot * 16 + lane
                for f in range(D):
                    fv = jnp.full((16,), f, jnp.int32)
                    v = plsc.load_gather(chunk_v, [fv, c16c])
                    plsc.store_scatter(stage_v, [rows16, fv], v)
                pos_v[slot] = jnp.where(inm, p16, -1)
                pltpu.async_copy(
                    stage_v.at[pl.ds(slot * 16, 16)],
                    out_hbm.at[plsc.Indices(pos_v.at[slot],
                                            ignored_value=-1)],
                    sem)
                return rc + 1

            return lax.cond(nhit > 0, do_extract, lambda rc: rc, rc)

        return lax.fori_loop(0, ngroups, grp_body, rc)

    rc = lax.fori_loop(0, nchunks, chunk_body, jnp.int32(0))

    def drain_body(k, _):
        @pl.when(k < jnp.minimum(rc, _RING))
        def _():
            pltpu.make_async_copy(
                out_hbm.at[pl.ds(0, 16)],
                stage_v.at[pl.ds(0, 16)], sem).wait()
        return 0

    lax.fori_loop(0, _RING, drain_body, 0)


@functools.partial(
    pl.kernel,
    out_type=[
        jax.ShapeDtypeStruct((B, 128), jnp.float32),
        jax.ShapeDtypeStruct((B, 128), jnp.float32),
    ],
    mesh=_sc_mesh,
    compiler_params=pltpu.CompilerParams(needs_layout_passes=False),
    scratch_types=[
        pltpu.VMEM((B,), jnp.int32),
        pltpu.VMEM((B + 16,), jnp.int32),
        pltpu.VMEM((B + 16,), jnp.int32),
        pltpu.VMEM((D, _CHUNK), jnp.float32),
        pltpu.VMEM((_RING * 16, 128), jnp.float32),
        pltpu.VMEM((_RING, 16), jnp.int32),
        pltpu.SemaphoreType.DMA,
    ],
)
def _sc_gather(u_fm_hbm, s_fm_hbm, uidx_hbm, sidx_hbm, uout_hbm, sout_hbm,
               idx_v, hcol_v, hpos_v, chunk_v, stage_v, pos_v, sem):
    wid = lax.axis_index("s") * _NC + lax.axis_index("c")

    pltpu.sync_copy(uidx_hbm, idx_v)
    _gather_table(u_fm_hbm, uout_hbm, idx_v, hcol_v, hpos_v, chunk_v,
                  stage_v, pos_v, sem, wid, _U_BLKS,
                  (_U_BLKS // _NW + 1) * 128 // _CHUNK + 1)
    pltpu.sync_copy(sidx_hbm, idx_v)
    _gather_table(s_fm_hbm, sout_hbm, idx_v, hcol_v, hpos_v, chunk_v,
                  stage_v, pos_v, sem, wid, _S_BLKS,
                  (_S_BLKS // _NW + 1) * 128 // _CHUNK + 1)


def _mlp_body(u_ref, s_ref, tvu_ref, tvs_ref, mu_ref, ms_ref, wt_ref,
              w1u_ref, w1s_ref, w1c_ref, b1_ref, w2_ref, b2_ref, w3_ref,
              b3_ref, out_ref):
    u = jnp.where(mu_ref[...] != 0, tvu_ref[...], u_ref[:, :D])
    s = jnp.where(ms_ref[...] != 0, tvs_ref[...], s_ref[:, :D])
    x = (
        jnp.dot(u, w1u_ref[...], preferred_element_type=jnp.float32)
        + jnp.dot(s, w1s_ref[...], preferred_element_type=jnp.float32)
        + jnp.dot(wt_ref[...], w1c_ref[...], preferred_element_type=jnp.float32)
        + b1_ref[...]
    )
    h = jnp.maximum(x, 0.0)
    h = jnp.dot(h, w2_ref[...], preferred_element_type=jnp.float32) + b2_ref[...]
    h = jnp.maximum(h, 0.0)
    o = jnp.dot(h, w3_ref[...], preferred_element_type=jnp.float32) + b3_ref[...]
    out_ref[...] = jax.nn.sigmoid(o)


_MLP_BLK = 2048


def _mlp(u, s, tvu, tvs, mu, ms, wt, w1u, w1s, w1c, b1, w2, b2, w3, b3):
    grid = (B // _MLP_BLK,)
    return pl.pallas_call(
        _mlp_body,
        grid=grid,
        in_specs=[
            pl.BlockSpec((_MLP_BLK, 128), lambda i: (i, 0)),
            pl.BlockSpec((_MLP_BLK, 128), lambda i: (i, 0)),
            pl.BlockSpec((_MLP_BLK, D), lambda i: (i, 0)),
            pl.BlockSpec((_MLP_BLK, D), lambda i: (i, 0)),
            pl.BlockSpec((_MLP_BLK, 1), lambda i: (i, 0)),
            pl.BlockSpec((_MLP_BLK, 1), lambda i: (i, 0)),
            pl.BlockSpec((_MLP_BLK, 2), lambda i: (i, 0)),
            pl.BlockSpec((D, H1), lambda i: (0, 0)),
            pl.BlockSpec((D, H1), lambda i: (0, 0)),
            pl.BlockSpec((2, H1), lambda i: (0, 0)),
            pl.BlockSpec((1, H1), lambda i: (0, 0)),
            pl.BlockSpec((H1, H2), lambda i: (0, 0)),
            pl.BlockSpec((1, H2), lambda i: (0, 0)),
            pl.BlockSpec((H2, 1), lambda i: (0, 0)),
            pl.BlockSpec((1, 1), lambda i: (0, 0)),
        ],
        out_specs=pl.BlockSpec((_MLP_BLK, 1), lambda i: (i, 0)),
        out_shape=jax.ShapeDtypeStruct((B, 1), jnp.float32),
    )(u, s, tvu, tvs, mu, ms, wt, w1u, w1s, w1c, b1, w2, b2, w3, b3)


@jax.jit
def kernel(user, song, weather, time, user_emb, song_emb, W1, b1, W2, b2, W3, b3):
    uidx = user.astype(jnp.int32)
    sidx = song.astype(jnp.int32)
    u_fm = user_emb.T
    s_fm = song_emb.T

    uidx_c = jnp.minimum(uidx, _U_BOUND - 1)
    sidx_c = jnp.minimum(sidx, _S_BOUND - 1)
    u_pad, s_pad = _sc_gather(u_fm, s_fm, uidx_c, sidx_c)

    mu = (uidx >= _U_BOUND).astype(jnp.float32)[:, None]
    ms = (sidx >= _S_BOUND).astype(jnp.float32)[:, None]
    tvu = jnp.take(user_emb[_U_BOUND:], jnp.clip(uidx - _U_BOUND, 0, NU - _U_BOUND - 1), axis=0)
    tvs = jnp.take(song_emb[_S_BOUND:], jnp.clip(sidx - _S_BOUND, 0, NS - _S_BOUND - 1), axis=0)

    wt = jnp.stack([weather, time], axis=1)
    w1u = W1[:D]
    w1s = W1[D:2 * D]
    w1c = W1[2 * D:]

    out = _mlp(u_pad, s_pad, tvu, tvs, mu, ms, wt, w1u, w1s, w1c,
               b1[None, :], W2, b2[None, :], W3, b3[None, :])
    return jnp.squeeze(out, axis=-1)


# speedup vs baseline: 2.1194x; 1.0650x over previous
import functools

import jax
import jax.numpy as jnp
from jax import lax
from jax.experimental import pallas as pl
from jax.experimental.pallas import tpu as pltpu
from jax.experimental.pallas import tpu_sc as plsc

B = 16384
D = 32
H1 = 64
H2 = 32
NU = 100000
NS = 1000000

_info = plsc.get_sparse_core_info()
_NC, _NSUB = _info.num_cores, _info.num_subcores
_NW = _NC * _NSUB

_CHUNK = 1280
_RING = 4
_PSH = 14

_U_BLKS = NU // 128
_S_BLKS = NS // 128
_U_BOUND = _U_BLKS * 128
_S_BOUND = _S_BLKS * 128

_sc_mesh = plsc.VectorSubcoreMesh(core_axis_name="c", subcore_axis_name="s")


def _nchunks(nblks_total):
    span = (nblks_total // _NW + 1) * 128
    return (span + _CHUNK - 1) // _CHUNK


def _gather_table(tbl_hbm, out_hbm, idx_v, hpk_v, chunk_v, stage_v,
                  pos_v, sem, fsem, wid, nblks_total, nchunks):
    nbase = nblks_total // _NW
    nrem = nblks_total - nbase * _NW
    nblk = nbase + jnp.where(wid < nrem, 1, 0)
    blk0 = wid * nbase + jnp.minimum(wid, nrem)
    lo = blk0 * 128
    hi = lo + nblk * 128

    lane = lax.iota(jnp.int32, 16)
    lo_v = jnp.zeros((16,), jnp.int32) + lo
    hi_v = jnp.zeros((16,), jnp.int32) + hi

    def chunk_start(c):
        return jnp.minimum(lo + c * _CHUNK, hi - _CHUNK)

    pltpu.async_copy(
        tbl_hbm.at[:, pl.ds(chunk_start(0), _CHUNK)], chunk_v.at[0], fsem)

    def pf_body(g, hcnt):
        h = hcnt
        for k in range(2):
            gg = g * 2 + k
            v16 = idx_v[pl.ds(gg * 16, 16)]
            m = (v16 >= lo_v) & (v16 < hi_v)
            pk = lax.shift_left(v16 - lo_v, jnp.int32(_PSH)) | (gg * 16 + lane)
            plsc.store_compressed(hpk_v.at[pl.ds(h, 16)], pk, mask=m)
            h = h + jnp.max(plsc.all_reduce_population_count(m))
        return h

    hcnt = lax.fori_loop(0, B // 32, pf_body, jnp.int32(0))
    hpk_v[pl.ds(hcnt, 16)] = jnp.full((16,), -1, jnp.int32)
    ngroups = (hcnt + 15) // 16

    def chunk_body(c, rc):
        par = lax.bitwise_and(c, jnp.int32(1))

        @pl.when(c + 1 < nchunks)
        def _():
            pltpu.async_copy(
                tbl_hbm.at[:, pl.ds(chunk_start(c + 1), _CHUNK)],
                chunk_v.at[1 - par], fsem)

        pltpu.make_async_copy(
            tbl_hbm.at[:, pl.ds(0, _CHUNK)], chunk_v.at[par], fsem).wait()

        crel = chunk_start(c) - lo
        a_v = jnp.zeros((16,), jnp.int32) + lax.shift_left(crel, jnp.int32(_PSH))
        b_v = jnp.zeros((16,), jnp.int32) + lax.shift_left(
            crel + _CHUNK, jnp.int32(_PSH))
        crel_v = jnp.zeros((16,), jnp.int32) + crel

        def grp_body(g, rc):
            h16 = hpk_v[pl.ds(g * 16, 16)]
            inm = (h16 >= a_v) & (h16 < b_v)
            nhit = jnp.max(plsc.all_reduce_population_count(inm))

            def do_extract(rc):
                slot = lax.bitwise_and(rc, jnp.int32(_RING - 1))

                @pl.when(rc >= _RING)
                def _():
                    pltpu.make_async_copy(
                        out_hbm.at[pl.ds(0, 16)],
                        stage_v.at[pl.ds(slot * 16, 16)], sem).wait()

                c16 = lax.shift_right_logical(h16, jnp.int32(_PSH)) - crel_v
                c16c = jnp.clip(c16, 0, _CHUNK - 1)
                p16 = lax.bitwise_and(h16, jnp.int32((1 << _PSH) - 1))
                rows16 = slot * 16 + lane
                par16 = jnp.zeros((16,), jnp.int32) + par
                for f in range(D):
                    fv = jnp.full((16,), f, jnp.int32)
                    v = plsc.load_gather(chunk_v, [par16, fv, c16c])
                    plsc.store_scatter(stage_v, [rows16, fv], v)
                pos_v[slot] = jnp.where(inm, p16, -1)
                pltpu.async_copy(
                    stage_v.at[pl.ds(slot * 16, 16)],
                    out_hbm.at[plsc.Indices(pos_v.at[slot],
                                            ignored_value=-1)],
                    sem)
                return rc + 1

            return lax.cond(nhit > 0, do_extract, lambda rc: rc, rc)

        return lax.fori_loop(0, ngroups, grp_body, rc)

    rc = lax.fori_loop(0, nchunks, chunk_body, jnp.int32(0))

    def drain_body(k, _):
        @pl.when(k < jnp.minimum(rc, _RING))
        def _():
            pltpu.make_async_copy(
                out_hbm.at[pl.ds(0, 16)],
                stage_v.at[pl.ds(0, 16)], sem).wait()
        return 0

    lax.fori_loop(0, _RING, drain_body, 0)


@functools.partial(
    pl.kernel,
    out_type=[
        jax.ShapeDtypeStruct((B, 128), jnp.float32),
        jax.ShapeDtypeStruct((B, 128), jnp.float32),
    ],
    mesh=_sc_mesh,
    compiler_params=pltpu.CompilerParams(needs_layout_passes=False),
    scratch_types=[
        pltpu.VMEM((B,), jnp.int32),
        pltpu.VMEM((B + 16,), jnp.int32),
        pltpu.VMEM((2, D, _CHUNK), jnp.float32),
        pltpu.VMEM((_RING * 16, 128), jnp.float32),
        pltpu.VMEM((_RING, 16), jnp.int32),
        pltpu.SemaphoreType.DMA,
        pltpu.SemaphoreType.DMA,
    ],
)
def _sc_gather(u_fm_hbm, s_fm_hbm, uidx_hbm, sidx_hbm, uout_hbm, sout_hbm,
               idx_v, hpk_v, chunk_v, stage_v, pos_v, sem, fsem):
    wid = lax.axis_index("s") * _NC + lax.axis_index("c")

    pltpu.sync_copy(uidx_hbm, idx_v)
    _gather_table(u_fm_hbm, uout_hbm, idx_v, hpk_v, chunk_v, stage_v,
                  pos_v, sem, fsem, wid, _U_BLKS, _nchunks(_U_BLKS))
    pltpu.sync_copy(sidx_hbm, idx_v)
    _gather_table(s_fm_hbm, sout_hbm, idx_v, hpk_v, chunk_v, stage_v,
                  pos_v, sem, fsem, wid, _S_BLKS, _nchunks(_S_BLKS))


def _mlp_body(u_ref, s_ref, tvu_ref, tvs_ref, mu_ref, ms_ref, wt_ref,
              w1u_ref, w1s_ref, w1c_ref, b1_ref, w2_ref, b2_ref, w3_ref,
              b3_ref, out_ref):
    u = jnp.where(mu_ref[...] != 0, tvu_ref[...], u_ref[:, :D])
    s = jnp.where(ms_ref[...] != 0, tvs_ref[...], s_ref[:, :D])
    x = (
        jnp.dot(u, w1u_ref[...], preferred_element_type=jnp.float32)
        + jnp.dot(s, w1s_ref[...], preferred_element_type=jnp.float32)
        + jnp.dot(wt_ref[...], w1c_ref[...], preferred_element_type=jnp.float32)
        + b1_ref[...]
    )
    h = jnp.maximum(x, 0.0)
    h = jnp.dot(h, w2_ref[...], preferred_element_type=jnp.float32) + b2_ref[...]
    h = jnp.maximum(h, 0.0)
    o = jnp.dot(h, w3_ref[...], preferred_element_type=jnp.float32) + b3_ref[...]
    out_ref[...] = jax.nn.sigmoid(o)


_MLP_BLK = 2048


def _mlp(u, s, tvu, tvs, mu, ms, wt, w1u, w1s, w1c, b1, w2, b2, w3, b3):
    grid = (B // _MLP_BLK,)
    return pl.pallas_call(
        _mlp_body,
        grid=grid,
        in_specs=[
            pl.BlockSpec((_MLP_BLK, 128), lambda i: (i, 0)),
            pl.BlockSpec((_MLP_BLK, 128), lambda i: (i, 0)),
            pl.BlockSpec((_MLP_BLK, D), lambda i: (i, 0)),
            pl.BlockSpec((_MLP_BLK, D), lambda i: (i, 0)),
            pl.BlockSpec((_MLP_BLK, 1), lambda i: (i, 0)),
            pl.BlockSpec((_MLP_BLK, 1), lambda i: (i, 0)),
            pl.BlockSpec((_MLP_BLK, 2), lambda i: (i, 0)),
            pl.BlockSpec((D, H1), lambda i: (0, 0)),
            pl.BlockSpec((D, H1), lambda i: (0, 0)),
            pl.BlockSpec((2, H1), lambda i: (0, 0)),
            pl.BlockSpec((1, H1), lambda i: (0, 0)),
            pl.BlockSpec((H1, H2), lambda i: (0, 0)),
            pl.BlockSpec((1, H2), lambda i: (0, 0)),
            pl.BlockSpec((H2, 1), lambda i: (0, 0)),
            pl.BlockSpec((1, 1), lambda i: (0, 0)),
        ],
        out_specs=pl.BlockSpec((_MLP_BLK, 1), lambda i: (i, 0)),
        out_shape=jax.ShapeDtypeStruct((B, 1), jnp.float32),
    )(u, s, tvu, tvs, mu, ms, wt, w1u, w1s, w1c, b1, w2, b2, w3, b3)


@jax.jit
def kernel(user, song, weather, time, user_emb, song_emb, W1, b1, W2, b2, W3, b3):
    uidx = user.astype(jnp.int32)
    sidx = song.astype(jnp.int32)
    u_fm = user_emb.T
    s_fm = song_emb.T

    uidx_c = jnp.minimum(uidx, _U_BOUND - 1)
    sidx_c = jnp.minimum(sidx, _S_BOUND - 1)
    u_pad, s_pad = _sc_gather(u_fm, s_fm, uidx_c, sidx_c)

    mu = (uidx >= _U_BOUND).astype(jnp.float32)[:, None]
    ms = (sidx >= _S_BOUND).astype(jnp.float32)[:, None]
    tvu = jnp.take(user_emb[_U_BOUND:], jnp.clip(uidx - _U_BOUND, 0, NU - _U_BOUND - 1), axis=0)
    tvs = jnp.take(song_emb[_S_BOUND:], jnp.clip(sidx - _S_BOUND, 0, NS - _S_BOUND - 1), axis=0)

    wt = jnp.stack([weather, time], axis=1)
    w1u = W1[:D]
    w1s = W1[D:2 * D]
    w1c = W1[2 * D:]

    out = _mlp(u_pad, s_pad, tvu, tvs, mu, ms, wt, w1u, w1s, w1c,
               b1[None, :], W2, b2[None, :], W3, b3[None, :])
    return jnp.squeeze(out, axis=-1)


# speedup vs baseline: 2.1398x; 1.0096x over previous
import functools

import jax
import jax.numpy as jnp
from jax import lax
from jax.experimental import pallas as pl
from jax.experimental.pallas import tpu as pltpu
from jax.experimental.pallas import tpu_sc as plsc

B = 16384
D = 32
H1 = 64
H2 = 32
NU = 100000
NS = 1000000

_info = plsc.get_sparse_core_info()
_NC, _NSUB = _info.num_cores, _info.num_subcores
_NW = _NC * _NSUB

_CHUNK = 1536
_IDXC = 4096
_RING = 4
_PSH = 14

_U_BLKS = NU // 128
_S_BLKS = NS // 128
_U_BOUND = _U_BLKS * 128
_S_BOUND = _S_BLKS * 128

_sc_mesh = plsc.VectorSubcoreMesh(core_axis_name="c", subcore_axis_name="s")


def _nchunks(nblks_total):
    span = (nblks_total // _NW + 1) * 128
    return (span + _CHUNK - 1) // _CHUNK


def _gather_table(tbl_hbm, out_hbm, idx_hbm, idx_v, hpk_v, chunk_v, stage_v,
                  pos_v, sem, fsem, wid, nblks_total, nchunks):
    nbase = nblks_total // _NW
    nrem = nblks_total - nbase * _NW
    nblk = nbase + jnp.where(wid < nrem, 1, 0)
    blk0 = wid * nbase + jnp.minimum(wid, nrem)
    lo = blk0 * 128
    hi = lo + nblk * 128

    lane = lax.iota(jnp.int32, 16)
    lo_v = jnp.zeros((16,), jnp.int32) + lo
    hi_v = jnp.zeros((16,), jnp.int32) + hi

    def chunk_start(c):
        return jnp.minimum(lo + c * _CHUNK, hi - _CHUNK)

    def fetch(c, par):
        s0 = chunk_start(c)
        pltpu.async_copy(
            tbl_hbm.at[pl.ds(0, 16), pl.ds(s0, _CHUNK)],
            chunk_v.at[par, pl.ds(0, 16)], fsem)
        pltpu.async_copy(
            tbl_hbm.at[pl.ds(16, 16), pl.ds(s0, _CHUNK)],
            chunk_v.at[par, pl.ds(16, 16)], fsem)

    fetch(0, 0)
    fetch(1, 1)

    def pf_stage(pbase, hcnt0):
        def pf_body(g, hcnt):
            h = hcnt
            for k in range(2):
                gg = g * 2 + k
                v16 = idx_v[pl.ds(gg * 16, 16)]
                m = (v16 >= lo_v) & (v16 < hi_v)
                pk = lax.shift_left(v16 - lo_v, jnp.int32(_PSH)) | (
                    pbase + gg * 16 + lane)
                plsc.store_compressed(hpk_v.at[pl.ds(h, 16)], pk, mask=m)
                h = h + jnp.max(plsc.all_reduce_population_count(m))
            return h
        return lax.fori_loop(0, _IDXC // 32, pf_body, hcnt0)

    with jax.named_scope("prefilter"):
        hcnt = jnp.int32(0)
        for pp in range(B // _IDXC):
            pltpu.sync_copy(idx_hbm.at[pl.ds(pp * _IDXC, _IDXC)], idx_v)
            hcnt = pf_stage(jnp.int32(pp * _IDXC), hcnt)
    hpk_v[pl.ds(hcnt, 16)] = jnp.full((16,), -1, jnp.int32)
    ngroups = (hcnt + 15) // 16

    def chunk_body(c, rc):
        par = lax.bitwise_and(c, jnp.int32(1))

        pltpu.make_async_copy(
            tbl_hbm.at[:, pl.ds(0, _CHUNK)], chunk_v.at[par], fsem).wait()


        crel = chunk_start(c) - lo
        a_v = jnp.zeros((16,), jnp.int32) + lax.shift_left(crel, jnp.int32(_PSH))
        b_v = jnp.zeros((16,), jnp.int32) + lax.shift_left(
            crel + _CHUNK, jnp.int32(_PSH))
        crel_v = jnp.zeros((16,), jnp.int32) + crel

        def grp_body(g, rc):
            h16 = hpk_v[pl.ds(g * 16, 16)]
            inm = (h16 >= a_v) & (h16 < b_v)
            nhit = jnp.any(inm)

            def do_extract(rc):
                slot = lax.bitwise_and(rc, jnp.int32(_RING - 1))

                @pl.when(rc >= _RING)
                def _():
                    pltpu.make_async_copy(
                        out_hbm.at[pl.ds(0, 16)],
                        stage_v.at[pl.ds(slot * 16, 16)], sem).wait()

                c16 = lax.shift_right_logical(h16, jnp.int32(_PSH)) - crel_v
                c16c = jnp.clip(c16, 0, _CHUNK - 1)
                p16 = lax.bitwise_and(h16, jnp.int32((1 << _PSH) - 1))
                rows16 = slot * 16 + lane
                par16 = jnp.zeros((16,), jnp.int32) + par
                for f in range(D):
                    fv = jnp.full((16,), f, jnp.int32)
                    v = plsc.load_gather(chunk_v, [par16, fv, c16c])
                    plsc.store_scatter(stage_v, [rows16, fv], v)
                pos_v[slot] = jnp.where(inm, p16, -1)
                pltpu.async_copy(
                    stage_v.at[pl.ds(slot * 16, 16)],
                    out_hbm.at[plsc.Indices(pos_v.at[slot],
                                            ignored_value=-1)],
                    sem)
                return rc + 1

            return lax.cond(nhit, do_extract, lambda rc: rc, rc)

        rc = lax.fori_loop(0, ngroups, grp_body, rc)

        @pl.when(c + 2 < nchunks)
        def _():
            fetch(c + 2, par)

        return rc

    with jax.named_scope("stream"):
        rc = lax.fori_loop(0, nchunks, chunk_body, jnp.int32(0))

    def drain_body(k, _):
        @pl.when(k < jnp.minimum(rc, _RING))
        def _():
            pltpu.make_async_copy(
                out_hbm.at[pl.ds(0, 16)],
                stage_v.at[pl.ds(0, 16)], sem).wait()
        return 0

    lax.fori_loop(0, _RING, drain_body, 0)


@functools.partial(
    pl.kernel,
    out_type=[
        jax.ShapeDtypeStruct((B, 128), jnp.float32),
        jax.ShapeDtypeStruct((B, 128), jnp.float32),
    ],
    mesh=_sc_mesh,
    compiler_params=pltpu.CompilerParams(needs_layout_passes=False),
    scratch_types=[
        pltpu.VMEM((_IDXC,), jnp.int32),
        pltpu.VMEM((B + 16,), jnp.int32),
        pltpu.VMEM((2, D, _CHUNK), jnp.float32),
        pltpu.VMEM((_RING * 16, 128), jnp.float32),
        pltpu.VMEM((_RING, 16), jnp.int32),
        pltpu.SemaphoreType.DMA,
        pltpu.SemaphoreType.DMA,
    ],
)
def _sc_gather(u_fm_hbm, s_fm_hbm, uidx_hbm, sidx_hbm, uout_hbm, sout_hbm,
               idx_v, hpk_v, chunk_v, stage_v, pos_v, sem, fsem):
    wid = lax.axis_index("s") * _NC + lax.axis_index("c")

    _gather_table(u_fm_hbm, uout_hbm, uidx_hbm, idx_v, hpk_v, chunk_v,
                  stage_v, pos_v, sem, fsem, wid, _U_BLKS, _nchunks(_U_BLKS))
    _gather_table(s_fm_hbm, sout_hbm, sidx_hbm, idx_v, hpk_v, chunk_v,
                  stage_v, pos_v, sem, fsem, wid, _S_BLKS, _nchunks(_S_BLKS))


def _mlp_body(u_ref, s_ref, uq_ref, sq_ref, tu_ref, ts_ref, wt_ref,
              w1u_ref, w1s_ref, w1c_ref, b1_ref, w2_ref, b2_ref, w3_ref,
              b3_ref, out_ref):
    uq = uq_ref[...]
    sq = sq_ref[...]
    iu = lax.broadcasted_iota(jnp.int32, (1, NU - _U_BOUND), 1).astype(jnp.float32)
    isv = lax.broadcasted_iota(jnp.int32, (1, NS - _S_BOUND), 1).astype(jnp.float32)
    ohu = (uq == iu).astype(jnp.float32)
    ohs = (sq == isv).astype(jnp.float32)
    tvu = jnp.dot(ohu, tu_ref[...], preferred_element_type=jnp.float32)
    tvs = jnp.dot(ohs, ts_ref[...], preferred_element_type=jnp.float32)
    u = jnp.where(uq >= 0, tvu, u_ref[:, :D])
    s = jnp.where(sq >= 0, tvs, s_ref[:, :D])
    x = (
        jnp.dot(u, w1u_ref[...], preferred_element_type=jnp.float32)
        + jnp.dot(s, w1s_ref[...], preferred_element_type=jnp.float32)
        + jnp.dot(wt_ref[...], w1c_ref[...], preferred_element_type=jnp.float32)
        + b1_ref[...]
    )
    h = jnp.maximum(x, 0.0)
    h = jnp.dot(h, w2_ref[...], preferred_element_type=jnp.float32) + b2_ref[...]
    h = jnp.maximum(h, 0.0)
    o = jnp.dot(h, w3_ref[...], preferred_element_type=jnp.float32) + b3_ref[...]
    out_ref[...] = jax.nn.sigmoid(o)


_MLP_BLK = 2048


def _mlp(u, s, uq, sq, tu, ts, wt, w1u, w1s, w1c, b1, w2, b2, w3, b3):
    grid = (B // _MLP_BLK,)
    return pl.pallas_call(
        _mlp_body,
        grid=grid,
        in_specs=[
            pl.BlockSpec((_MLP_BLK, 128), lambda i: (i, 0)),
            pl.BlockSpec((_MLP_BLK, 128), lambda i: (i, 0)),
            pl.BlockSpec((_MLP_BLK, 1), lambda i: (i, 0)),
            pl.BlockSpec((_MLP_BLK, 1), lambda i: (i, 0)),
            pl.BlockSpec((NU - _U_BOUND, D), lambda i: (0, 0)),
            pl.BlockSpec((NS - _S_BOUND, D), lambda i: (0, 0)),
            pl.BlockSpec((_MLP_BLK, 2), lambda i: (i, 0)),
            pl.BlockSpec((D, H1), lambda i: (0, 0)),
            pl.BlockSpec((D, H1), lambda i: (0, 0)),
            pl.BlockSpec((2, H1), lambda i: (0, 0)),
            pl.BlockSpec((1, H1), lambda i: (0, 0)),
            pl.BlockSpec((H1, H2), lambda i: (0, 0)),
            pl.BlockSpec((1, H2), lambda i: (0, 0)),
            pl.BlockSpec((H2, 1), lambda i: (0, 0)),
            pl.BlockSpec((1, 1), lambda i: (0, 0)),
        ],
        out_specs=pl.BlockSpec((_MLP_BLK, 1), lambda i: (i, 0)),
        out_shape=jax.ShapeDtypeStruct((B, 1), jnp.float32),
    )(u, s, uq, sq, tu, ts, wt, w1u, w1s, w1c, b1, w2, b2, w3, b3)


@jax.jit
def kernel(user, song, weather, time, user_emb, song_emb, W1, b1, W2, b2, W3, b3):
    uidx = user.astype(jnp.int32)
    sidx = song.astype(jnp.int32)
    u_fm = user_emb.T
    s_fm = song_emb.T

    uidx_c = jnp.minimum(uidx, _U_BOUND - 1)
    sidx_c = jnp.minimum(sidx, _S_BOUND - 1)
    u_pad, s_pad = _sc_gather(u_fm, s_fm, uidx_c, sidx_c)

    uq = jnp.where(uidx >= _U_BOUND, uidx - _U_BOUND, -1).astype(jnp.float32)[:, None]
    sq = jnp.where(sidx >= _S_BOUND, sidx - _S_BOUND, -1).astype(jnp.float32)[:, None]
    tu = user_emb[_U_BOUND:]
    ts = song_emb[_S_BOUND:]

    wt = jnp.stack([weather, time], axis=1)
    w1u = W1[:D]
    w1s = W1[D:2 * D]
    w1c = W1[2 * D:]

    out = _mlp(u_pad, s_pad, uq, sq, tu, ts, wt, w1u, w1s, w1c,
               b1[None, :], W2, b2[None, :], W3, b3[None, :])
    return jnp.squeeze(out, axis=-1)
